# Initial kernel scaffold; baseline (speedup 1.0000x reference)
import jax
import jax.numpy as jnp
from jax.experimental import pallas as pl


def kernel(x, router_w, router_b, w1, b1, w2, b2):
    raise NotImplementedError("write your pallas kernel here")


# speedup vs baseline: 3.0679x; 3.0679x over previous
import jax
import jax.numpy as jnp
from jax.experimental import pallas as pl
from jax.experimental.pallas import tpu as pltpu

D_MODEL = 768
D_FF = 3072
E = 8
TOPK = 2
T = 2048
F_BLK = 1024
NF = D_FF // F_BLK

_NEG = float("-inf")


def _router_body(x_ref, rw_ref, rb_ref, c_ref, aux_ref):
    x = x_ref[...]
    logits = jnp.dot(x, rw_ref[...], preferred_element_type=jnp.float32)
    logits = logits + rb_ref[...]
    eidx = jax.lax.broadcasted_iota(jnp.int32, (T, E), 1)
    m1 = jnp.max(logits, axis=1, keepdims=True)
    i1 = jnp.min(jnp.where(logits == m1, eidx, E), axis=1, keepdims=True)
    mask1 = eidx == i1
    l2 = jnp.where(mask1, _NEG, logits)
    m2 = jnp.max(l2, axis=1, keepdims=True)
    i2 = jnp.min(jnp.where(l2 == m2, eidx, E), axis=1, keepdims=True)
    mask2 = eidx == i2
    sel = mask1 | mask2
    z = jnp.exp(logits - m1)
    w = z / jnp.sum(z, axis=1, keepdims=True)
    wsel = jnp.where(sel, w, 0.0)
    c_ref[...] = wsel / jnp.sum(wsel, axis=1, keepdims=True)
    cnt = jnp.sum(sel.astype(jnp.float32), axis=0, keepdims=True)
    util = cnt / jnp.float32(T * TOPK)
    mu = jnp.mean(util)
    var = jnp.sum((util - mu) ** 2) / jnp.float32(E - 1)
    cv = jnp.sqrt(var) / (mu + 1e-6)
    aux_ref[...] = jnp.broadcast_to(cv * cv, (1, 1))


def _expert_body(c_ref, x_ref, w1_ref, b1_ref, w2_ref, b2_ref, out_ref):
    e = pl.program_id(0)
    f = pl.program_id(1)
    x = x_ref[...]
    h = jnp.dot(x, w1_ref[0], preferred_element_type=jnp.float32) + b1_ref[0]
    h = 0.5 * h * (1.0 + jax.lax.erf(h * 0.7071067811865476))
    y = jnp.dot(h, w2_ref[0], preferred_element_type=jnp.float32)
    eidx = jax.lax.broadcasted_iota(jnp.int32, (T, E), 1)
    ce = jnp.sum(jnp.where(eidx == e, c_ref[...], 0.0), axis=1, keepdims=True)

    @pl.when((e == 0) & (f == 0))
    def _init():
        out_ref[...] = jnp.zeros_like(out_ref)

    @pl.when(f == 0)
    def _bias():
        out_ref[...] += ce * b2_ref[0]

    out_ref[...] += ce * y


def _router(xf, router_w, router_b, interpret=False):
    return pl.pallas_call(
        _router_body,
        out_shape=[
            jax.ShapeDtypeStruct((T, E), jnp.float32),
            jax.ShapeDtypeStruct((1, 1), jnp.float32),
        ],
        interpret=interpret,
    )(xf, router_w, router_b.reshape(1, E))


def _experts(c, xf, w1, b1, w2, b2, interpret=False):
    return pl.pallas_call(
        _expert_body,
        grid=(E, NF),
        in_specs=[
            pl.BlockSpec((T, E), lambda e, f: (0, 0)),
            pl.BlockSpec((T, D_MODEL), lambda e, f: (0, 0)),
            pl.BlockSpec((1, D_MODEL, F_BLK), lambda e, f: (e, 0, f)),
            pl.BlockSpec((1, 1, F_BLK), lambda e, f: (e, 0, f)),
            pl.BlockSpec((1, F_BLK, D_MODEL), lambda e, f: (e, f, 0)),
            pl.BlockSpec((1, 1, D_MODEL), lambda e, f: (e, 0, 0)),
        ],
        out_specs=pl.BlockSpec((T, D_MODEL), lambda e, f: (0, 0)),
        out_shape=jax.ShapeDtypeStruct((T, D_MODEL), jnp.float32),
        compiler_params=pltpu.CompilerParams(
            dimension_semantics=("arbitrary", "arbitrary"),
        ),
        interpret=interpret,
    )(c, xf, w1, b1.reshape(E, 1, D_FF), w2, b2.reshape(E, 1, D_MODEL))


def kernel(x, router_w, router_b, w1, b1, w2, b2):
    orig_shape = x.shape
    xf = x.reshape(-1, D_MODEL)
    c, aux = _router(xf, router_w, router_b)
    out = _experts(c, xf, w1, b1, w2, b2)
    return out.reshape(orig_shape), aux[0, 0]
